# Initial kernel scaffold; baseline (speedup 1.0000x reference)
import jax
import jax.numpy as jnp
from jax.experimental import pallas as pl


def kernel(source_subtoken_indices, node_indices, target_subtoken_indices, source_subtoken_lengths, node_lengths, target_subtoken_lengths, context_valid_mask, subtoken_embedding, node_embedding, Wx_f, Wh_f, b_f, Wx_b, Wh_b, b_b, W_ctx):
    raise NotImplementedError("write your pallas kernel here")


# speedup vs baseline: 4.0094x; 4.0094x over previous
import functools

import jax
import jax.numpy as jnp
from jax import lax
from jax.experimental import pallas as pl
from jax.experimental.pallas import tpu as pltpu
from jax.experimental.pallas import tpu_sc as plsc

B, C, S, L = 64, 200, 6, 9
D_TOK, D_NODE, H, D_DEC = 128, 128, 128, 512
NODE_VOCAB = 512
N = B * C
NCTX = 2 * N
CHUNK_CTX = 16
ROWS_PER_CHUNK = CHUNK_CTX * S
N_CHUNKS = NCTX // CHUNK_CTX
NW = 32
CHUNKS_PER_W = N_CHUNKS // NW
LANES = 16
TILE = 256


def _sc_pool_body(idx_hbm, mask_hbm, table_hbm, out_hbm, idx_v, mask_v, rows_v, acc_v, sem):
    wid = lax.axis_index("s") * 2 + lax.axis_index("c")

    def chunk_body(j, _):
        chunk = wid * CHUNKS_PER_W + j
        pltpu.sync_copy(idx_hbm.at[chunk], idx_v)
        pltpu.sync_copy(mask_hbm.at[chunk], mask_v)
        pltpu.async_copy(table_hbm.at[idx_v], rows_v, sem).wait()
        for ci in range(CHUNK_CTX):
            ms = [mask_v[ci * S + s, :] for s in range(S)]
            for v in range(D_TOK // LANES):
                acc = rows_v[ci * S + 0, pl.ds(v * LANES, LANES)] * ms[0]
                for s in range(1, S):
                    acc = acc + rows_v[ci * S + s, pl.ds(v * LANES, LANES)] * ms[s]
                acc_v[ci, pl.ds(v * LANES, LANES)] = acc
        pltpu.sync_copy(acc_v, out_hbm.at[pl.ds(chunk * CHUNK_CTX, CHUNK_CTX)])
        return 0

    lax.fori_loop(0, CHUNKS_PER_W, chunk_body, 0)


def _sc_pool_call(idx_cat, mask_cat, table):
    mesh = plsc.VectorSubcoreMesh(core_axis_name="c", subcore_axis_name="s")
    fn = functools.partial(
        pl.kernel,
        mesh=mesh,
        out_type=jax.ShapeDtypeStruct((NCTX, D_TOK), jnp.float32),
        scratch_types=[
            pltpu.VMEM((ROWS_PER_CHUNK,), jnp.int32),
            pltpu.VMEM((ROWS_PER_CHUNK, LANES), jnp.float32),
            pltpu.VMEM((ROWS_PER_CHUNK, D_TOK), jnp.float32),
            pltpu.VMEM((CHUNK_CTX, D_TOK), jnp.float32),
            pltpu.SemaphoreType.DMA,
        ],
    )(_sc_pool_body)
    return fn(idx_cat, mask_cat, table)


def _prep_body(emb_ref, wxf_ref, bf_ref, wxb_ref, bb_ref, tf_ref, tb_ref):
    emb = emb_ref[...]
    tf_ref[...] = jnp.dot(emb, wxf_ref[...], preferred_element_type=jnp.float32) + bf_ref[...]
    tb_ref[...] = jnp.dot(emb, wxb_ref[...], preferred_element_type=jnp.float32) + bb_ref[...]


def _prep_call(node_embedding, Wx_f, b_f, Wx_b, b_b):
    return pl.pallas_call(
        _prep_body,
        out_shape=(
            jax.ShapeDtypeStruct((NODE_VOCAB, 4 * H), jnp.float32),
            jax.ShapeDtypeStruct((NODE_VOCAB, 4 * H), jnp.float32),
        ),
    )(node_embedding, Wx_f, b_f.reshape(1, 4 * H), Wx_b, b_b.reshape(1, 4 * H))


def _lstm_body(idxf_ref, idxb_ref, len_ref, sa_ref, ta_ref, cvm_ref,
               tf_ref, tb_ref, whf_ref, whb_ref, wctx_ref, out_ref):
    f32 = jnp.float32
    lenc = len_ref[...]
    tf = tf_ref[...]
    tb = tb_ref[...]
    whf = whf_ref[...]
    whb = whb_ref[...]
    iota = lax.broadcasted_iota(jnp.int32, (TILE, NODE_VOCAB), 1)

    hf = jnp.zeros((TILE, H), f32)
    cf = jnp.zeros((TILE, H), f32)
    hb = jnp.zeros((TILE, H), f32)
    cb = jnp.zeros((TILE, H), f32)

    def cell(idx_t, h, c, t_tab, wh, mask):
        oh = (idx_t == iota).astype(f32)
        gates = (jnp.dot(oh, t_tab, preferred_element_type=f32)
                 + jnp.dot(h, wh, preferred_element_type=f32))
        i = jax.nn.sigmoid(gates[:, 0:H])
        f = jax.nn.sigmoid(gates[:, H:2 * H])
        g = jnp.tanh(gates[:, 2 * H:3 * H])
        o = jax.nn.sigmoid(gates[:, 3 * H:4 * H])
        c_new = f * c + i * g
        h_new = o * jnp.tanh(c_new)
        return jnp.where(mask, h_new, h), jnp.where(mask, c_new, c)

    for t in range(L):
        mask = t < lenc
        hf, cf = cell(idxf_ref[:, t:t + 1], hf, cf, tf, whf, mask)
        hb, cb = cell(idxb_ref[:, t:t + 1], hb, cb, tb, whb, mask)

    cvm = cvm_ref[...]
    wctx = wctx_ref[...]
    out = (jnp.dot(sa_ref[...], wctx[0:D_TOK], preferred_element_type=f32)
           + jnp.dot(hf * cvm, wctx[D_TOK:D_TOK + H], preferred_element_type=f32)
           + jnp.dot(hb * cvm, wctx[D_TOK + H:D_TOK + 2 * H], preferred_element_type=f32)
           + jnp.dot(ta_ref[...], wctx[D_TOK + 2 * H:], preferred_element_type=f32))
    out_ref[...] = jnp.tanh(out)


def _lstm_call(idxf, idxb, lens, src_agg, tgt_agg, cvm, tf, tb, whf, whb, wctx):
    grid = (N // TILE,)
    row = lambda i: (i, 0)
    rep = lambda i: (0, 0)
    return pl.pallas_call(
        _lstm_body,
        grid=grid,
        in_specs=[
            pl.BlockSpec((TILE, L), row),
            pl.BlockSpec((TILE, L), row),
            pl.BlockSpec((TILE, 1), row),
            pl.BlockSpec((TILE, D_TOK), row),
            pl.BlockSpec((TILE, D_TOK), row),
            pl.BlockSpec((TILE, 1), row),
            pl.BlockSpec((NODE_VOCAB, 4 * H), rep),
            pl.BlockSpec((NODE_VOCAB, 4 * H), rep),
            pl.BlockSpec((H, 4 * H), rep),
            pl.BlockSpec((H, 4 * H), rep),
            pl.BlockSpec((2 * (D_TOK + H), D_DEC), rep),
        ],
        out_specs=pl.BlockSpec((TILE, D_DEC), row),
        out_shape=jax.ShapeDtypeStruct((N, D_DEC), jnp.float32),
    )(idxf, idxb, lens, src_agg, tgt_agg, cvm, tf, tb, whf, whb, wctx)


def kernel(source_subtoken_indices, node_indices, target_subtoken_indices,
           source_subtoken_lengths, node_lengths, target_subtoken_lengths,
           context_valid_mask, subtoken_embedding, node_embedding,
           Wx_f, Wh_f, b_f, Wx_b, Wh_b, b_b, W_ctx):
    src_idx = source_subtoken_indices.reshape(N, S)
    tgt_idx = target_subtoken_indices.reshape(N, S)
    idx_cat = jnp.concatenate([src_idx, tgt_idx], axis=0).reshape(N_CHUNKS, ROWS_PER_CHUNK)
    ar = jnp.arange(S)[None, :]
    src_mask = (ar < source_subtoken_lengths.reshape(N, 1)).astype(jnp.float32)
    tgt_mask = (ar < target_subtoken_lengths.reshape(N, 1)).astype(jnp.float32)
    mask_cat = jnp.concatenate([src_mask, tgt_mask], axis=0).reshape(NCTX * S, 1)
    mask_cat = jnp.broadcast_to(mask_cat, (NCTX * S, LANES)).reshape(
        N_CHUNKS, ROWS_PER_CHUNK, LANES)

    nidx = node_indices.reshape(N, L)
    lens = node_lengths.reshape(N)
    jrev = jnp.clip(lens[:, None] - 1 - jnp.arange(L)[None, :], 0, L - 1)
    idxb = jnp.take_along_axis(nidx, jrev, axis=1)

    pooled = _sc_pool_call(idx_cat, mask_cat, subtoken_embedding)
    src_agg = pooled[:N]
    tgt_agg = pooled[N:]

    tf, tb = _prep_call(node_embedding, Wx_f, b_f, Wx_b, b_b)
    out = _lstm_call(nidx, idxb, lens.reshape(N, 1), src_agg, tgt_agg,
                     context_valid_mask.reshape(N, 1), tf, tb, Wh_f, Wh_b, W_ctx)
    return out.reshape(B, C, D_DEC)


# speedup vs baseline: 5.0269x; 1.2538x over previous
import functools

import jax
import jax.numpy as jnp
from jax import lax
from jax.experimental import pallas as pl
from jax.experimental.pallas import tpu as pltpu
from jax.experimental.pallas import tpu_sc as plsc

B, C, S, L = 64, 200, 6, 9
D_TOK, D_NODE, H, D_DEC = 128, 128, 128, 512
NODE_VOCAB = 512
N = B * C
NCTX = 2 * N
CHUNK_CTX = 16
ROWS_PER_CHUNK = CHUNK_CTX * S
N_CHUNKS = NCTX // CHUNK_CTX
NW = 32
CHUNKS_PER_W = N_CHUNKS // NW
LANES = 16
TILE = 256
G4 = 4 * H


def _sc_pool_body(idx_hbm, mask_hbm, table_hbm, out_hbm, idx_v, mask_v, rows_v, acc_v, sem):
    wid = lax.axis_index("s") * 2 + lax.axis_index("c")

    def chunk_body(j, _):
        chunk = wid * CHUNKS_PER_W + j
        pltpu.sync_copy(idx_hbm.at[chunk], idx_v)
        pltpu.sync_copy(mask_hbm.at[chunk], mask_v)
        pltpu.async_copy(table_hbm.at[idx_v], rows_v, sem).wait()
        for ci in range(CHUNK_CTX):
            ms = [mask_v[ci * S + s, :] for s in range(S)]
            for v in range(D_TOK // LANES):
                acc = rows_v[ci * S + 0, pl.ds(v * LANES, LANES)] * ms[0]
                for s in range(1, S):
                    acc = acc + rows_v[ci * S + s, pl.ds(v * LANES, LANES)] * ms[s]
                acc_v[ci, pl.ds(v * LANES, LANES)] = acc
        pltpu.sync_copy(acc_v, out_hbm.at[pl.ds(chunk * CHUNK_CTX, CHUNK_CTX)])
        return 0

    lax.fori_loop(0, CHUNKS_PER_W, chunk_body, 0)


def _sc_pool_call(idx_cat, mask_cat, table):
    mesh = plsc.VectorSubcoreMesh(core_axis_name="c", subcore_axis_name="s")
    fn = functools.partial(
        pl.kernel,
        mesh=mesh,
        out_type=jax.ShapeDtypeStruct((NCTX, D_TOK), jnp.float32),
        scratch_types=[
            pltpu.VMEM((ROWS_PER_CHUNK,), jnp.int32),
            pltpu.VMEM((ROWS_PER_CHUNK, LANES), jnp.float32),
            pltpu.VMEM((ROWS_PER_CHUNK, D_TOK), jnp.float32),
            pltpu.VMEM((CHUNK_CTX, D_TOK), jnp.float32),
            pltpu.SemaphoreType.DMA,
        ],
    )(_sc_pool_body)
    return fn(idx_cat, mask_cat, table)


def _prep_body(emb_ref, wxf_ref, bf_ref, wxb_ref, bb_ref, t_ref):
    emb = emb_ref[...]
    tf = jnp.dot(emb, wxf_ref[...], preferred_element_type=jnp.float32) + bf_ref[...]
    tb = jnp.dot(emb, wxb_ref[...], preferred_element_type=jnp.float32) + bb_ref[...]
    t_ref[...] = jnp.concatenate([tf, tb], axis=1).astype(jnp.bfloat16)


def _prep_call(node_embedding, Wx_f, b_f, Wx_b, b_b):
    return pl.pallas_call(
        _prep_body,
        out_shape=jax.ShapeDtypeStruct((NODE_VOCAB, 2 * G4), jnp.bfloat16),
    )(node_embedding, Wx_f, b_f.reshape(1, G4), Wx_b, b_b.reshape(1, G4))


def _sigmoid(x):
    return 0.5 + 0.5 * jnp.tanh(0.5 * x)


def _lstm_body(idx_ref, len_ref, t_ref, wh_ref, out_ref, oh_ref, g_ref):
    f32 = jnp.float32
    idx = idx_ref[...]
    iota = lax.broadcasted_iota(jnp.int32, (TILE, NODE_VOCAB), 1)
    for p in range(L):
        oh_ref[pl.ds(p * TILE, TILE), :] = (idx[:, p:p + 1] == iota).astype(jnp.bfloat16)
    g_ref[...] = jnp.dot(oh_ref[...], t_ref[...], preferred_element_type=f32)

    lenc = len_ref[...]
    wh = wh_ref[...]
    hf = jnp.zeros((TILE, H), f32)
    cf = jnp.zeros((TILE, H), f32)
    hb = jnp.zeros((TILE, H), f32)
    cb = jnp.zeros((TILE, H), f32)

    def cell(gates, h, c, mask):
        i = _sigmoid(gates[:, 0:H])
        f = _sigmoid(gates[:, H:2 * H])
        g = jnp.tanh(gates[:, 2 * H:3 * H])
        o = _sigmoid(gates[:, 3 * H:4 * H])
        c_new = f * c + i * g
        h_new = o * jnp.tanh(c_new)
        return jnp.where(mask, h_new, h), jnp.where(mask, c_new, c)

    for t in range(L):
        hcat = jnp.concatenate([hf, hb], axis=1).astype(jnp.bfloat16)
        rec = jnp.dot(hcat, wh, preferred_element_type=f32)
        gf = g_ref[pl.ds(t * TILE, TILE), 0:G4] + rec[:, 0:G4]
        gb = g_ref[pl.ds((L - 1 - t) * TILE, TILE), G4:2 * G4] + rec[:, G4:2 * G4]
        hf, cf = cell(gf, hf, cf, t < lenc)
        hb, cb = cell(gb, hb, cb, (L - 1 - t) < lenc)

    out_ref[...] = jnp.concatenate([hf, hb], axis=1).astype(jnp.bfloat16)


def _lstm_call(idx, lens, tcat, whcat):
    row = lambda i: (i, 0)
    rep = lambda i: (0, 0)
    return pl.pallas_call(
        _lstm_body,
        grid=(N // TILE,),
        in_specs=[
            pl.BlockSpec((TILE, L), row),
            pl.BlockSpec((TILE, 1), row),
            pl.BlockSpec((NODE_VOCAB, 2 * G4), rep),
            pl.BlockSpec((2 * H, 2 * G4), rep),
        ],
        out_specs=pl.BlockSpec((TILE, 2 * H), row),
        out_shape=jax.ShapeDtypeStruct((N, 2 * H), jnp.bfloat16),
        scratch_shapes=[
            pltpu.VMEM((L * TILE, NODE_VOCAB), jnp.bfloat16),
            pltpu.VMEM((L * TILE, 2 * G4), jnp.float32),
        ],
    )(idx, lens, tcat, whcat)


def _gemm_body(sa_ref, ta_ref, h_ref, cvm_ref, w_ref, out_ref):
    f32 = jnp.float32
    bf16 = jnp.bfloat16
    w = w_ref[...]
    cvm = cvm_ref[...].astype(bf16)
    out = (jnp.dot(sa_ref[...].astype(bf16), w[0:D_TOK], preferred_element_type=f32)
           + jnp.dot(h_ref[...] * cvm, w[D_TOK:D_TOK + 2 * H], preferred_element_type=f32)
           + jnp.dot(ta_ref[...].astype(bf16), w[D_TOK + 2 * H:], preferred_element_type=f32))
    out_ref[...] = jnp.tanh(out)


def _gemm_call(src_agg, tgt_agg, hcat, cvm, wctx):
    row = lambda i: (i, 0)
    rep = lambda i: (0, 0)
    return pl.pallas_call(
        _gemm_body,
        grid=(N // TILE,),
        in_specs=[
            pl.BlockSpec((TILE, D_TOK), row),
            pl.BlockSpec((TILE, D_TOK), row),
            pl.BlockSpec((TILE, 2 * H), row),
            pl.BlockSpec((TILE, 1), row),
            pl.BlockSpec((2 * (D_TOK + H), D_DEC), rep),
        ],
        out_specs=pl.BlockSpec((TILE, D_DEC), row),
        out_shape=jax.ShapeDtypeStruct((N, D_DEC), jnp.float32),
    )(src_agg, tgt_agg, hcat, cvm, wctx)


def kernel(source_subtoken_indices, node_indices, target_subtoken_indices,
           source_subtoken_lengths, node_lengths, target_subtoken_lengths,
           context_valid_mask, subtoken_embedding, node_embedding,
           Wx_f, Wh_f, b_f, Wx_b, Wh_b, b_b, W_ctx):
    src_idx = source_subtoken_indices.reshape(N, S)
    tgt_idx = target_subtoken_indices.reshape(N, S)
    idx_cat = jnp.concatenate([src_idx, tgt_idx], axis=0).reshape(N_CHUNKS, ROWS_PER_CHUNK)
    ar = jnp.arange(S)[None, :]
    src_mask = (ar < source_subtoken_lengths.reshape(N, 1)).astype(jnp.float32)
    tgt_mask = (ar < target_subtoken_lengths.reshape(N, 1)).astype(jnp.float32)
    mask_cat = jnp.concatenate([src_mask, tgt_mask], axis=0).reshape(NCTX * S, 1)
    mask_cat = jnp.broadcast_to(mask_cat, (NCTX * S, LANES)).reshape(
        N_CHUNKS, ROWS_PER_CHUNK, LANES)

    nidx = node_indices.reshape(N, L)
    lens = node_lengths.reshape(N, 1)
    whcat = jnp.zeros((2 * H, 2 * G4), jnp.float32)
    whcat = whcat.at[0:H, 0:G4].set(Wh_f).at[H:2 * H, G4:2 * G4].set(Wh_b)
    whcat = whcat.astype(jnp.bfloat16)
    wctx = W_ctx.astype(jnp.bfloat16)

    pooled = _sc_pool_call(idx_cat, mask_cat, subtoken_embedding)

    tcat = _prep_call(node_embedding, Wx_f, b_f, Wx_b, b_b)
    hcat = _lstm_call(nidx, lens, tcat, whcat)
    out = _gemm_call(pooled[:N], pooled[N:], hcat,
                     context_valid_mask.reshape(N, 1), wctx)
    return out.reshape(B, C, D_DEC)
